# Initial kernel scaffold; baseline (speedup 1.0000x reference)
import jax
import jax.numpy as jnp
from jax.experimental import pallas as pl


def kernel(q_feat, k_feat, q_pos, k_pos, Wq, bq, Wk, bk, Wv, bv, W1, b1, W2, b2):
    raise NotImplementedError("write your pallas kernel here")


# speedup vs baseline: 9.3207x; 9.3207x over previous
import functools

import jax
import jax.numpy as jnp
from jax import lax
from jax.experimental import pallas as pl
from jax.experimental.pallas import tpu as pltpu
from jax.experimental.pallas import tpu_sc as plsc

B, S, N, DIM, K = 4, 256, 16384, 256, 16
CW = 2048
NCH = N // CW
BSK = B * S * K
PW = 128
NW = 32
RPW = BSK // NW
GCH = 128


def _knn_body(qp_ref, kpt_ref, idx_ref, dist_ref):
    b = pl.program_id(0)
    qp = qp_ref[0]
    q2 = jnp.sum(qp * qp, axis=1, keepdims=True)

    def build(c, _):
        kc = kpt_ref[0, :, pl.dslice(c * CW, CW)]
        qk = lax.dot_general(qp, kc, (((1,), (0,)), ((), ())),
                             preferred_element_type=jnp.float32)
        k2 = jnp.sum(kc * kc, axis=0, keepdims=True)
        dist_ref[:, pl.dslice(c * CW, CW)] = q2 + k2 - 2.0 * qk
        return 0

    lax.fori_loop(0, NCH, build, 0)

    big_i = jnp.int32(N)
    kiota = lax.broadcasted_iota(jnp.int32, (S, K), 1)

    def select(t, carry):
        dprev, iprev, acc = carry

        def scan_chunk(c, mc):
            m, am = mc
            d = dist_ref[:, pl.dslice(c * CW, CW)]
            gi = lax.broadcasted_iota(jnp.int32, (S, CW), 1) + c * CW
            elig = (d > dprev) | ((d == dprev) & (gi > iprev))
            de = jnp.where(elig, d, jnp.inf)
            cm = jnp.min(de, axis=1, keepdims=True)
            ca = jnp.min(jnp.where(de == cm, gi, big_i), axis=1,
                         keepdims=True)
            take = cm < m
            return jnp.where(take, cm, m), jnp.where(take, ca, am)

        m0 = jnp.full((S, 1), jnp.inf, jnp.float32)
        a0 = jnp.full((S, 1), big_i, jnp.int32)
        m, am = lax.fori_loop(0, NCH, scan_chunk, (m0, a0))
        acc = jnp.where(kiota == t, am, acc)
        return m, am, acc

    d0 = jnp.full((S, 1), -jnp.inf, jnp.float32)
    i0 = jnp.full((S, 1), jnp.int32(-1), jnp.int32)
    acc0 = jnp.zeros((S, K), jnp.int32)
    _, _, acc = lax.fori_loop(0, K, select, (d0, i0, acc0))
    idx_ref[0] = acc + b * N


def _knn(qp8, kpt8):
    return pl.pallas_call(
        _knn_body,
        grid=(B,),
        in_specs=[
            pl.BlockSpec((1, S, 8), lambda b: (b, 0, 0)),
            pl.BlockSpec((1, 8, N), lambda b: (b, 0, 0)),
        ],
        out_specs=pl.BlockSpec((1, S, K), lambda b: (b, 0, 0)),
        out_shape=jax.ShapeDtypeStruct((B, S, K), jnp.int32),
        scratch_shapes=[pltpu.VMEM((S, N), jnp.float32)],
    )(qp8, kpt8)


def _sc_gather(kf2d, kp2d, idx_flat):
    mesh = plsc.VectorSubcoreMesh(core_axis_name="c", subcore_axis_name="s")

    @functools.partial(
        pl.kernel, mesh=mesh,
        out_type=[
            jax.ShapeDtypeStruct((BSK, DIM), jnp.float32),
            jax.ShapeDtypeStruct((BSK, PW), jnp.float32),
        ],
        scratch_types=[
            pltpu.VMEM((RPW,), jnp.int32),
            pltpu.VMEM((GCH, DIM), jnp.float32),
            pltpu.VMEM((GCH, PW), jnp.float32),
            pltpu.SemaphoreType.DMA,
        ],
    )
    def gather(kf_hbm, kp_hbm, idx_hbm, okf_hbm, okp_hbm,
               idx_v, rows_v, kpr_v, sem):
        wid = lax.axis_index("s") * 2 + lax.axis_index("c")
        base = wid * RPW
        pltpu.sync_copy(idx_hbm.at[pl.dslice(base, RPW)], idx_v)
        for j in range(RPW // GCH):
            idx_j = idx_v.at[pl.dslice(j * GCH, GCH)]
            pltpu.async_copy(kf_hbm.at[idx_j], rows_v, sem).wait()
            pltpu.sync_copy(rows_v,
                            okf_hbm.at[pl.dslice(base + j * GCH, GCH)])
            pltpu.async_copy(kp_hbm.at[idx_j], kpr_v, sem).wait()
            pltpu.sync_copy(kpr_v,
                            okp_hbm.at[pl.dslice(base + j * GCH, GCH)])

    return gather(kf2d, kp2d, idx_flat)


def _epi_body(kf_ref, kp_ref, qf_ref, qp_ref, wq_ref, wk_ref, wv_ref,
              w1_ref, w2_ref, bq_ref, bk_ref, bv_ref, b1_ref, b2_ref,
              out_ref):
    kf = kf_ref[0]
    kp = kp_ref[0]
    qf = qf_ref[0]
    qp = qp_ref[0]

    wqs = jnp.sum(wq_ref[...], axis=0, keepdims=True)
    wks = jnp.sum(wk_ref[...], axis=0, keepdims=True)
    w2s = jnp.sum(w2_ref[...], axis=0, keepdims=True)
    bqs = jnp.sum(bq_ref[...])
    bks = jnp.sum(bk_ref[...])
    b2s = jnp.sum(b2_ref[...])

    dot = functools.partial(lax.dot_general,
                            preferred_element_type=jnp.float32)
    cN = (((1,), (1,)), ((), ()))
    c0 = (((1,), (0,)), ((), ()))

    qsum = jnp.sum(qf * wqs, axis=1, keepdims=True) + bqs
    ksum = jnp.sum(kf * wks, axis=1, keepdims=True) + bks

    qpw = dot(qp, w1_ref[:8, :], c0)
    kpw = dot(kp, w1_ref[...], c0)
    h3 = jnp.maximum(
        qpw[:, None, :] - kpw.reshape(S, K, DIM) + b1_ref[...][None], 0.0)
    possum = jnp.sum(h3.reshape(S * K, DIM) * w2s, axis=1,
                     keepdims=True) + b2s

    z2 = (possum - ksum).reshape(S, K)
    logits = z2 + qsum
    mx = jnp.max(logits, axis=1, keepdims=True)
    e = jnp.exp(logits - mx)
    attn_n = e / jnp.sum(e, axis=1, keepdims=True)
    colsum = jnp.sum(attn_n, axis=0, keepdims=True)
    attn = attn_n / (colsum + 1e-6)
    attn_k = jnp.sum(attn, axis=1, keepdims=True)

    wkf = attn.reshape(S * K, 1) * kf
    agg = jnp.sum(wkf.reshape(S, K, DIM), axis=1)
    out_ref[0] = dot(agg, wv_ref[...], cN) + attn_k * bv_ref[...]


def _epilogue(kf_nb, kp_nb, q_feat, qp8, Wq, Wk, Wv, W1T16, W2,
              bq, bk, bv, b1, b2):
    full = lambda s: pl.BlockSpec(s, lambda b: tuple(0 for _ in s))
    return pl.pallas_call(
        _epi_body,
        grid=(B,),
        in_specs=[
            pl.BlockSpec((1, S * K, DIM), lambda b: (b, 0, 0)),
            pl.BlockSpec((1, S * K, PW), lambda b: (b, 0, 0)),
            pl.BlockSpec((1, S, DIM), lambda b: (b, 0, 0)),
            pl.BlockSpec((1, S, 8), lambda b: (b, 0, 0)),
            full((DIM, DIM)), full((DIM, DIM)), full((DIM, DIM)),
            full((PW, DIM)), full((DIM, DIM)),
            full((1, DIM)), full((1, DIM)), full((1, DIM)),
            full((1, DIM)), full((1, DIM)),
        ],
        out_specs=pl.BlockSpec((1, S, DIM), lambda b: (b, 0, 0)),
        out_shape=jax.ShapeDtypeStruct((B, S, DIM), jnp.float32),
    )(kf_nb, kp_nb, q_feat, qp8, Wq, Wk, Wv, W1T16, W2,
      bq, bk, bv, b1, b2)


def kernel(q_feat, k_feat, q_pos, k_pos, Wq, bq, Wk, bk, Wv, bv,
           W1, b1, W2, b2):
    f32 = jnp.float32
    qp8 = jnp.pad(q_pos, ((0, 0), (0, 0), (0, 5)))
    kpt8 = jnp.pad(jnp.swapaxes(k_pos, 1, 2), ((0, 0), (0, 5), (0, 0)))
    kp2d = jnp.pad(k_pos, ((0, 0), (0, 0), (0, PW - 3))).reshape(B * N, PW)
    kf2d = k_feat.reshape(B * N, DIM)
    w1t = jnp.pad(W1.T, ((0, PW - 3), (0, 0)))

    idx = _knn(qp8, kpt8)
    kf_nb, kp_nb = _sc_gather(kf2d, kp2d, idx.reshape(BSK))

    out = _epilogue(
        kf_nb.reshape(B, S * K, DIM), kp_nb.reshape(B, S * K, PW),
        q_feat, qp8, Wq, Wk, Wv, w1t, W2,
        bq.reshape(1, DIM).astype(f32), bk.reshape(1, DIM),
        bv.reshape(1, DIM), b1.reshape(1, DIM), b2.reshape(1, DIM))
    return out
